# baseline (speedup 1.0000x reference)
import functools

import jax
import jax.numpy as jnp
from jax import lax
from jax.experimental import pallas as pl
from jax.experimental.pallas import tpu as pltpu
from jax.experimental.pallas import tpu_sc as plsc

_EPS = 1e-5
_NC, _NS, _LANES = 2, 16, 16
_NW = _NC * _NS
_CHUNK = 256
_NBUF = 2
_D = 64


def _layernorm_chunk(in_ref, out_ref, idx_ref, srow, g_ref, b_ref):
  inv_d = jnp.float32(1.0 / _D)
  lane = lax.iota(jnp.int32, _LANES)
  out_row_half = (lane & 1) * _D
  nv = _D // _LANES
  gs = [g_ref[pl.ds(k * _LANES, _LANES)] for k in range(nv)]
  bs = [b_ref[pl.ds(k * _LANES, _LANES)] for k in range(nv)]

  @plsc.parallel_loop(0, _CHUNK, _LANES)
  def _group(r0):
    rows = r0 + lane
    orow = rows >> 1
    pbase = (idx_ref[srow + (r0 >> 7), pl.ds(r0 & 127, _LANES)] & 1) * _D
    acc0 = jnp.zeros((_LANES,), jnp.float32)
    acc1 = jnp.zeros((_LANES,), jnp.float32)
    sq0 = jnp.zeros((_LANES,), jnp.float32)
    sq1 = jnp.zeros((_LANES,), jnp.float32)
    for c in range(0, _D, 2):
      v0 = plsc.load_gather(in_ref, [rows, pbase + c])
      v1 = plsc.load_gather(in_ref, [rows, pbase + (c + 1)])
      acc0 = acc0 + v0
      sq0 = sq0 + v0 * v0
      acc1 = acc1 + v1
      sq1 = sq1 + v1 * v1
    mean = (acc0 + acc1) * inv_d
    var = (sq0 + sq1) * inv_d - mean * mean
    v0_ = var + jnp.float32(_EPS)
    y = plsc.bitcast(
        jnp.full((_LANES,), jnp.int32(0x5F3759DF)) -
        (plsc.bitcast(v0_, jnp.int32) >> 1), jnp.float32)
    half_v = jnp.float32(0.5) * v0_
    for _ in range(3):
      y = y * (jnp.float32(1.5) - half_v * y * y)
    for c in range(_D):
      v = plsc.load_gather(in_ref, [rows, pbase + c])
      o = (v - mean) * y * gs[c // _LANES][c % _LANES] \
          + bs[c // _LANES][c % _LANES]
      plsc.store_scatter(out_ref, [orow, out_row_half + c], o)


def _sc_body(x_hbm, table_hbm, gamma_hbm, beta_hbm, out_hbm,
             idx_v0, idx_v1, in_v0, in_v1, out_v0, out_v1, g_v, b_v,
             gsem0, gsem1, osem0, osem1, isem0, isem1):
  idx_vs = (idx_v0, idx_v1)
  in_vs = (in_v0, in_v1)
  out_vs = (out_v0, out_v1)
  gsems = (gsem0, gsem1)
  osems = (osem0, osem1)
  isems = (isem0, isem1)

  n_rows = x_hbm.shape[0] * x_hbm.shape[1]
  rows_tile = n_rows // _NW
  n_chunks = rows_tile // _CHUNK

  wid = lax.axis_index("s") * _NC + lax.axis_index("c")
  r_base = wid * rows_tile

  pltpu.sync_copy(gamma_hbm, g_v)
  pltpu.sync_copy(beta_hbm, b_v)

  def srow_of(g):
    return ((r_base + g * _CHUNK) // 128) & 7

  def idx_copy(g, b):
    base = pl.multiple_of(((r_base + g * _CHUNK) // 128) & ~7, 8)
    return pltpu.make_async_copy(
        x_hbm.at[pl.ds(base, 8)], idx_vs[b], isems[b])

  def fire_gathers(g, b):
    srow = srow_of(g)
    for r0 in range(0, _CHUNK, _LANES):
      lvec = idx_vs[b][srow + r0 // 128, pl.ds(r0 % 128, _LANES)] >> 1
      pltpu.async_copy(
          table_hbm.at[lvec], in_vs[b].at[pl.ds(r0, _LANES), :], gsems[b])

  def drain_gathers(b):
    pltpu.make_async_copy(
        table_hbm.at[pl.ds(0, _CHUNK), :], in_vs[b], gsems[b]).wait()

  def out_copy(g, b):
    base = pl.multiple_of((r_base + g * _CHUNK) // 2, 8)
    return pltpu.make_async_copy(
        out_vs[b], out_hbm.at[pl.ds(base, _CHUNK // 2)], osems[b])

  for b in range(_NBUF):
    idx_copy(b, b).start()
    idx_copy(b, b).wait()
    fire_gathers(b, b)

  @pl.loop(0, n_chunks, step=_NBUF)
  def _chunks(g0):
    for b in range(_NBUF):
      g = g0 + b
      nxt = g + _NBUF
      drain_gathers(b)

      @pl.when(nxt < n_chunks)
      def _prefetch_idx():
        idx_copy(nxt, b).start()

      @pl.when(g >= _NBUF)
      def _drain_out():
        out_copy(g - _NBUF, b).wait()

      _layernorm_chunk(in_vs[b], out_vs[b], idx_vs[b], srow_of(g), g_v, b_v)
      out_copy(g, b).start()

      @pl.when(nxt < n_chunks)
      def _fire_gathers():
        idx_copy(nxt, b).wait()
        fire_gathers(nxt, b)

  for b in range(_NBUF):
    out_copy(n_chunks - _NBUF + b, b).wait()


def kernel(x, table, gamma, beta):
  batch, seq = x.shape
  n = batch * seq
  d = table.shape[1]
  x2 = x.reshape(n // 128, 128).astype(jnp.int32)
  table2 = table.reshape(table.shape[0] // 2, 2 * d)

  mesh = plsc.VectorSubcoreMesh(core_axis_name="c", subcore_axis_name="s")
  run = pl.kernel(
      _sc_body,
      out_type=jax.ShapeDtypeStruct((n // 2, 2 * d), jnp.float32),
      mesh=mesh,
      compiler_params=pltpu.CompilerParams(
          needs_layout_passes=False, use_tc_tiling_on_sc=True),
      scratch_types=(
          [pltpu.VMEM((8, 128), jnp.int32)] * _NBUF
          + [pltpu.VMEM((_CHUNK, 2 * _D), jnp.float32)] * _NBUF
          + [pltpu.VMEM((_CHUNK // 2, 2 * _D), jnp.float32)] * _NBUF
          + [pltpu.VMEM((_D,), jnp.float32)] * 2
          + [pltpu.SemaphoreType.DMA] * (_NBUF * 3)
      ),
  )
  out = run(x2, table2, gamma, beta)
  return out.reshape(batch, seq, d)


# speedup vs baseline: 2.6687x; 2.6687x over previous
import functools

import jax
import jax.numpy as jnp
from jax import lax
from jax.experimental import pallas as pl
from jax.experimental.pallas import tpu as pltpu
from jax.experimental.pallas import tpu_sc as plsc

_EPS = 1e-5
_NC, _NS, _LANES = 2, 16, 16
_NW = _NC * _NS
_GROUP = 256
_GPC = 1
_CHUNK = _GROUP * _GPC
_NBUF = 4
_D = 64


def _layernorm_chunk(in_ref, out_ref, g_ref, b_ref):
  inv_d = jnp.float32(1.0 / _D)
  nv = _D // _LANES
  gs = [g_ref[pl.ds(k * _LANES, _LANES)] for k in range(nv)]
  bs = [b_ref[pl.ds(k * _LANES, _LANES)] for k in range(nv)]

  @plsc.parallel_loop(0, _CHUNK, 1, unroll=8)
  def _row(r):
    xs = [in_ref[r, pl.ds(k * _LANES, _LANES)] for k in range(nv)]
    tot = jnp.sum((xs[0] + xs[1]) + (xs[2] + xs[3]))
    mean = tot * inv_d
    ds = [x - mean for x in xs]
    q = (ds[0] * ds[0] + ds[1] * ds[1]) + (ds[2] * ds[2] + ds[3] * ds[3])
    v0 = jnp.sum(q) * inv_d + jnp.float32(_EPS)
    y = lax.bitcast_convert_type(
        jnp.int32(0x5F3759DF) - (lax.bitcast_convert_type(v0, jnp.int32) >> 1),
        jnp.float32)
    half_v = jnp.float32(0.5) * v0
    for _ in range(2):
      y = y * (jnp.float32(1.5) - half_v * y * y)
    for k in range(nv):
      out_ref[r, pl.ds(k * _LANES, _LANES)] = ds[k] * y * gs[k] + bs[k]


def _sc_body(x_hbm, table_hbm, gamma_hbm, beta_hbm, out_hbm,
             idx_v0, idx_v1, idx_v2, idx_v3,
             in_v0, in_v1, in_v2, in_v3,
             out_v0, out_v1, g_v, b_v,
             gsem0, gsem1, gsem2, gsem3,
             osem0, osem1, isem0, isem1, isem2, isem3):
  idx_vs = (idx_v0, idx_v1, idx_v2, idx_v3)
  in_vs = (in_v0, in_v1, in_v2, in_v3)
  out_vs = (out_v0, out_v1)
  gsems = (gsem0, gsem1, gsem2, gsem3)
  osems = (osem0, osem1)
  isems = (isem0, isem1, isem2, isem3)

  n_groups_total = x_hbm.shape[0]
  gp_tile = n_groups_total // _NW
  n_chunks = gp_tile // _GPC
  rows_tile = gp_tile * _GROUP

  wid = lax.axis_index("s") * _NC + lax.axis_index("c")
  g_base = wid * gp_tile
  r_base = wid * rows_tile

  pltpu.sync_copy(gamma_hbm, g_v)
  pltpu.sync_copy(beta_hbm, b_v)

  def idx_copy(g, b):
    return pltpu.make_async_copy(
        x_hbm.at[pl.ds(g_base + g * _GPC, _GPC)], idx_vs[b], isems[b])

  def fire_gathers(b):
    for r0 in range(0, _CHUNK, _LANES):
      ivec = idx_vs[b][0, pl.ds(r0, _LANES)]
      pltpu.async_copy(
          table_hbm.at[ivec], in_vs[b].at[pl.ds(r0, _LANES), :], gsems[b])

  def drain_gathers(b):
    pltpu.make_async_copy(
        table_hbm.at[pl.ds(0, _CHUNK), :], in_vs[b], gsems[b]).wait()

  def out_copy(g, b):
    return pltpu.make_async_copy(
        out_vs[b], out_hbm.at[pl.ds(r_base + g * _CHUNK, _CHUNK)], osems[b])

  for b in range(_NBUF):
    idx_copy(b, b).start()
    idx_copy(b, b).wait()
    fire_gathers(b)

  @pl.loop(0, n_chunks, step=_NBUF)
  def _chunks(g0):
    for b in range(_NBUF):
      g = g0 + b
      nxt = g + _NBUF
      drain_gathers(b)

      @pl.when(nxt < n_chunks)
      def _prefetch_idx():
        idx_copy(nxt, b).start()

      @pl.when(g >= 2)
      def _drain_out():
        out_copy(g - 2, b % 2).wait()

      _layernorm_chunk(in_vs[b], out_vs[b % 2], g_v, b_v)
      out_copy(g, b % 2).start()

      @pl.when(nxt < n_chunks)
      def _fire_gathers():
        idx_copy(nxt, b).wait()
        fire_gathers(b)

  for b in range(2):
    out_copy(n_chunks - 2 + b, (n_chunks - 2 + b) % 2).wait()


def kernel(x, table, gamma, beta):
  batch, seq = x.shape
  n = batch * seq
  d = table.shape[1]
  x2 = x.reshape(n // _GROUP, _GROUP).astype(jnp.int32)

  mesh = plsc.VectorSubcoreMesh(core_axis_name="c", subcore_axis_name="s")
  run = pl.kernel(
      _sc_body,
      out_type=jax.ShapeDtypeStruct((n, d), jnp.float32),
      mesh=mesh,
      compiler_params=pltpu.CompilerParams(
          needs_layout_passes=False, use_tc_tiling_on_sc=False),
      scratch_types=(
          [pltpu.VMEM((_GPC, _GROUP), jnp.int32)] * _NBUF
          + [pltpu.VMEM((_CHUNK, _D), jnp.float32)] * _NBUF
          + [pltpu.VMEM((_CHUNK, _D), jnp.float32)] * 2
          + [pltpu.VMEM((_D,), jnp.float32)] * 2
          + [pltpu.SemaphoreType.DMA] * (_NBUF * 2 + 2)
      ),
  )
  out = run(x2, table, gamma, beta)
  return out.reshape(batch, seq, d)


# speedup vs baseline: 2.6745x; 1.0022x over previous
import jax
import jax.numpy as jnp
from jax import lax
from jax.experimental import pallas as pl
from jax.experimental.pallas import tpu as pltpu
from jax.experimental.pallas import tpu_sc as plsc

_EPS = 1e-5
_NC, _NS, _LANES = 2, 16, 16
_NW = _NC * _NS
_GROUP = 256
_GPC = 1
_CHUNK = _GROUP * _GPC
_NBUF = 4
_D = 64


def _layernorm_chunk(in_ref, out_ref, g_ref, b_ref):
  inv_d = jnp.float32(1.0 / _D)
  nv = _D // _LANES
  gs = [g_ref[pl.ds(k * _LANES, _LANES)] for k in range(nv)]
  bs = [b_ref[pl.ds(k * _LANES, _LANES)] for k in range(nv)]

  @plsc.parallel_loop(0, _CHUNK, 1, unroll=8)
  def _row(r):
    xs = [in_ref[r, pl.ds(k * _LANES, _LANES)] for k in range(nv)]
    tot = jnp.sum((xs[0] + xs[1]) + (xs[2] + xs[3]))
    mean = tot * inv_d
    ds = [x - mean for x in xs]
    q = (ds[0] * ds[0] + ds[1] * ds[1]) + (ds[2] * ds[2] + ds[3] * ds[3])
    v0 = jnp.sum(q) * inv_d + jnp.float32(_EPS)
    y = lax.bitcast_convert_type(
        jnp.int32(0x5F3759DF) - (lax.bitcast_convert_type(v0, jnp.int32) >> 1),
        jnp.float32)
    half_v = jnp.float32(0.5) * v0
    for _ in range(2):
      y = y * (jnp.float32(1.5) - half_v * y * y)
    for k in range(nv):
      out_ref[r, pl.ds(k * _LANES, _LANES)] = ds[k] * y * gs[k] + bs[k]


def _sc_body(x_hbm, table_hbm, gamma_hbm, beta_hbm, out_hbm,
             idx_v0, idx_v1, idx_v2, idx_v3,
             in_v0, in_v1, in_v2, in_v3,
             out_v0, out_v1, g_v, b_v,
             gsem0, gsem1, gsem2, gsem3,
             osem0, osem1, isem0, isem1, isem2, isem3):
  idx_vs = (idx_v0, idx_v1, idx_v2, idx_v3)
  in_vs = (in_v0, in_v1, in_v2, in_v3)
  out_vs = (out_v0, out_v1)
  gsems = (gsem0, gsem1, gsem2, gsem3)
  osems = (osem0, osem1)
  isems = (isem0, isem1, isem2, isem3)

  n_groups_total = x_hbm.shape[0]
  gp_tile = n_groups_total // _NW
  n_chunks = gp_tile // _GPC
  rows_tile = gp_tile * _GROUP

  wid = lax.axis_index("s") * _NC + lax.axis_index("c")
  g_base = wid * gp_tile
  r_base = wid * rows_tile

  pltpu.sync_copy(gamma_hbm, g_v)
  pltpu.sync_copy(beta_hbm, b_v)

  def idx_copy(g, b):
    return pltpu.make_async_copy(
        x_hbm.at[pl.ds(g_base + g * _GPC, _GPC)], idx_vs[b], isems[b])

  def fire_gathers(b):
    for r0 in range(0, _CHUNK, _LANES):
      ivec = idx_vs[b][0, pl.ds(r0, _LANES)]
      pltpu.async_copy(
          table_hbm.at[ivec], in_vs[b].at[pl.ds(r0, _LANES), :], gsems[b])

  def drain_gathers(b):
    pltpu.make_async_copy(
        table_hbm.at[pl.ds(0, _CHUNK), :], in_vs[b], gsems[b]).wait()

  def out_copy(g, b):
    return pltpu.make_async_copy(
        out_vs[b], out_hbm.at[pl.ds(r_base + g * _CHUNK, _CHUNK)], osems[b])

  for b in range(_NBUF):
    idx_copy(b, b).start()
    idx_copy(b, b).wait()
    fire_gathers(b)

  @pl.loop(0, n_chunks, step=_NBUF)
  def _chunks(g0):
    for b in range(_NBUF):
      g = g0 + b
      nxt = g + _NBUF
      drain_gathers(b)

      @pl.when(nxt < n_chunks)
      def _prefetch_idx():
        idx_copy(nxt, b).start()

      @pl.when(g >= 2)
      def _drain_out():
        out_copy(g - 2, b % 2).wait()

      _layernorm_chunk(in_vs[b], out_vs[b % 2], g_v, b_v)
      out_copy(g, b % 2).start()

      @pl.when(nxt < n_chunks)
      def _fire_gathers():
        idx_copy(nxt, b).wait()
        fire_gathers(b)

  for b in range(2):
    out_copy(n_chunks - 2 + b, (n_chunks - 2 + b) % 2).wait()


def kernel(x, table, gamma, beta):
  batch, seq = x.shape
  n = batch * seq
  d = table.shape[1]
  x2 = x.reshape(n // _GROUP, _GROUP).astype(jnp.int32)

  mesh = plsc.VectorSubcoreMesh(core_axis_name="c", subcore_axis_name="s")
  run = pl.kernel(
      _sc_body,
      out_type=jax.ShapeDtypeStruct((n, d), jnp.float32),
      mesh=mesh,
      compiler_params=pltpu.CompilerParams(
          needs_layout_passes=False, use_tc_tiling_on_sc=False),
      scratch_types=(
          [pltpu.VMEM((_GPC, _GROUP), jnp.int32)] * _NBUF
          + [pltpu.VMEM((_CHUNK, _D), jnp.float32)] * _NBUF
          + [pltpu.VMEM((_CHUNK, _D), jnp.float32)] * 2
          + [pltpu.VMEM((_D,), jnp.float32)] * 2
          + [pltpu.SemaphoreType.DMA] * (_NBUF * 2 + 2)
      ),
  )
  out = run(x2, table, gamma, beta)
  return out.reshape(batch, seq, d)


# speedup vs baseline: 2.6766x; 1.0008x over previous
import jax
import jax.numpy as jnp
from jax import lax
from jax.experimental import pallas as pl
from jax.experimental.pallas import tpu as pltpu
from jax.experimental.pallas import tpu_sc as plsc

_EPS = 1e-5
_NC, _NS, _LANES = 2, 16, 16
_NW = _NC * _NS
_GROUP = 128
_GPC = 2
_CHUNK = _GROUP * _GPC
_NBUF = 2
_D = 64


def _layernorm_chunk(in_ref, out_ref, g_ref, b_ref):
  inv_d = jnp.float32(1.0 / _D)
  nv = _D // _LANES
  gs = [g_ref[pl.ds(k * _LANES, _LANES)] for k in range(nv)]
  bs = [b_ref[pl.ds(k * _LANES, _LANES)] for k in range(nv)]

  @plsc.parallel_loop(0, _CHUNK, 1, unroll=8)
  def _row(r):
    xs = [in_ref[r, pl.ds(k * _LANES, _LANES)] for k in range(nv)]
    tot = jnp.sum((xs[0] + xs[1]) + (xs[2] + xs[3]))
    mean = tot * inv_d
    ds = [x - mean for x in xs]
    q = (ds[0] * ds[0] + ds[1] * ds[1]) + (ds[2] * ds[2] + ds[3] * ds[3])
    v0 = jnp.sum(q) * inv_d + jnp.float32(_EPS)
    y = lax.bitcast_convert_type(
        jnp.int32(0x5F3759DF) - (lax.bitcast_convert_type(v0, jnp.int32) >> 1),
        jnp.float32)
    half_v = jnp.float32(0.5) * v0
    for _ in range(2):
      y = y * (jnp.float32(1.5) - half_v * y * y)
    for k in range(nv):
      out_ref[r, pl.ds(k * _LANES, _LANES)] = ds[k] * y * gs[k] + bs[k]


def _sc_body(x_hbm, table_hbm, gamma_hbm, beta_hbm, out_hbm,
             idx_v0, idx_v1, in_v0, in_v1, out_v0, out_v1, g_v, b_v,
             gsem0, gsem1, osem0, osem1, isem0, isem1):
  idx_vs = (idx_v0, idx_v1)
  in_vs = (in_v0, in_v1)
  out_vs = (out_v0, out_v1)
  gsems = (gsem0, gsem1)
  osems = (osem0, osem1)
  isems = (isem0, isem1)

  n_groups_total = x_hbm.shape[0]
  gp_tile = n_groups_total // _NW
  n_chunks = gp_tile // _GPC
  rows_tile = gp_tile * _GROUP

  wid = lax.axis_index("s") * _NC + lax.axis_index("c")
  g_base = wid * gp_tile
  r_base = wid * rows_tile

  pltpu.sync_copy(gamma_hbm, g_v)
  pltpu.sync_copy(beta_hbm, b_v)

  def idx_copy(g, b):
    return pltpu.make_async_copy(
        x_hbm.at[pl.ds(g_base + g * _GPC, _GPC)], idx_vs[b], isems[b])

  def gather_copy(b, j):
    return pltpu.make_async_copy(
        table_hbm.at[idx_vs[b].at[j]],
        in_vs[b].at[pl.ds(j * _GROUP, _GROUP), :], gsems[b])

  def out_copy(g, b):
    return pltpu.make_async_copy(
        out_vs[b], out_hbm.at[pl.ds(r_base + g * _CHUNK, _CHUNK)], osems[b])

  for b in range(_NBUF):
    idx_copy(b, b).start()
    idx_copy(b, b).wait()
    for j in range(_GPC):
      gather_copy(b, j).start()

  @pl.loop(0, n_chunks, step=_NBUF)
  def _chunks(g0):
    for b in range(_NBUF):
      g = g0 + b
      nxt = g + _NBUF
      for j in range(_GPC):
        gather_copy(b, j).wait()

      @pl.when(nxt < n_chunks)
      def _prefetch_idx():
        idx_copy(nxt, b).start()

      @pl.when(g >= _NBUF)
      def _drain_out():
        out_copy(g - _NBUF, b).wait()

      _layernorm_chunk(in_vs[b], out_vs[b], g_v, b_v)
      out_copy(g, b).start()

      @pl.when(nxt < n_chunks)
      def _fire_gathers():
        idx_copy(nxt, b).wait()
        for j in range(_GPC):
          gather_copy(b, j).start()

  for b in range(_NBUF):
    out_copy(n_chunks - _NBUF + b, b).wait()


def kernel(x, table, gamma, beta):
  batch, seq = x.shape
  n = batch * seq
  d = table.shape[1]
  x2 = x.reshape(n // _GROUP, _GROUP).astype(jnp.int32)

  mesh = plsc.VectorSubcoreMesh(core_axis_name="c", subcore_axis_name="s")
  run = pl.kernel(
      _sc_body,
      out_type=jax.ShapeDtypeStruct((n, d), jnp.float32),
      mesh=mesh,
      compiler_params=pltpu.CompilerParams(
          needs_layout_passes=False, use_tc_tiling_on_sc=False),
      scratch_types=[
          pltpu.VMEM((_GPC, _GROUP), jnp.int32),
          pltpu.VMEM((_GPC, _GROUP), jnp.int32),
          pltpu.VMEM((_CHUNK, _D), jnp.float32),
          pltpu.VMEM((_CHUNK, _D), jnp.float32),
          pltpu.VMEM((_CHUNK, _D), jnp.float32),
          pltpu.VMEM((_CHUNK, _D), jnp.float32),
          pltpu.VMEM((_D,), jnp.float32),
          pltpu.VMEM((_D,), jnp.float32),
          pltpu.SemaphoreType.DMA,
          pltpu.SemaphoreType.DMA,
          pltpu.SemaphoreType.DMA,
          pltpu.SemaphoreType.DMA,
          pltpu.SemaphoreType.DMA,
          pltpu.SemaphoreType.DMA,
      ],
  )
  out = run(x2, table, gamma, beta)
  return out.reshape(batch, seq, d)


# speedup vs baseline: 2.8756x; 1.0744x over previous
import jax
import jax.numpy as jnp
from jax import lax
from jax.experimental import pallas as pl
from jax.experimental.pallas import tpu as pltpu
from jax.experimental.pallas import tpu_sc as plsc

_EPS = 1e-5
_NC, _NS, _LANES = 2, 16, 16
_NW = _NC * _NS
_GROUP = 128
_GPC = 2
_CHUNK = _GROUP * _GPC
_NBUF = 2
_D = 64


def _layernorm_chunk(in_ref, out_ref, g_ref, b_ref):
  inv_d = jnp.float32(1.0 / _D)
  nv = _D // _LANES
  gs = [g_ref[pl.ds(k * _LANES, _LANES)] for k in range(nv)]
  bs = [b_ref[pl.ds(k * _LANES, _LANES)] for k in range(nv)]

  @plsc.parallel_loop(0, _CHUNK, 1, unroll=8)
  def _row(r):
    xs = [in_ref[r, pl.ds(k * _LANES, _LANES)] for k in range(nv)]
    tot = jnp.sum((xs[0] + xs[1]) + (xs[2] + xs[3]))
    mean = tot * inv_d
    ds = [x - mean for x in xs]
    q = (ds[0] * ds[0] + ds[1] * ds[1]) + (ds[2] * ds[2] + ds[3] * ds[3])
    v0 = jnp.sum(q) * inv_d + jnp.float32(_EPS)
    y = lax.bitcast_convert_type(
        jnp.int32(0x5F3759DF) - (lax.bitcast_convert_type(v0, jnp.int32) >> 1),
        jnp.float32)
    half_v = jnp.float32(0.5) * v0
    for _ in range(3):
      y = y * (jnp.float32(1.5) - half_v * y * y)
    for k in range(nv):
      out_ref[r, pl.ds(k * _LANES, _LANES)] = ds[k] * y * gs[k] + bs[k]


def _sc_body(x_hbm, table_hbm, gamma_hbm, beta_hbm, out_hbm,
             idx_v0, idx_v1, in_v0, in_v1, out_v0, out_v1, g_v, b_v,
             gsem0, gsem1, osem0, osem1, isem0, isem1):
  idx_vs = (idx_v0, idx_v1)
  in_vs = (in_v0, in_v1)
  out_vs = (out_v0, out_v1)
  gsems = (gsem0, gsem1)
  osems = (osem0, osem1)
  isems = (isem0, isem1)

  n_groups_total = x_hbm.shape[0]
  gp_tile = n_groups_total // _NW
  n_chunks = gp_tile // _GPC
  rows_tile = gp_tile * _GROUP

  wid = lax.axis_index("s") * _NC + lax.axis_index("c")
  g_base = wid * gp_tile
  r_base = wid * rows_tile

  pltpu.sync_copy(gamma_hbm, g_v)
  pltpu.sync_copy(beta_hbm, b_v)

  def idx_copy(g, b):
    return pltpu.make_async_copy(
        x_hbm.at[pl.ds(g_base + g * _GPC, _GPC)], idx_vs[b], isems[b])

  def gather_copy(b, j):
    return pltpu.make_async_copy(
        table_hbm.at[idx_vs[b].at[j]],
        in_vs[b].at[pl.ds(j * _GROUP, _GROUP), :], gsems[b])

  def out_copy(g, b):
    return pltpu.make_async_copy(
        out_vs[b], out_hbm.at[pl.ds(r_base + g * _CHUNK, _CHUNK)], osems[b])

  for b in range(_NBUF):
    idx_copy(b, b).start()
    idx_copy(b, b).wait()
    for j in range(_GPC):
      gather_copy(b, j).start()

  @pl.loop(0, n_chunks, step=_NBUF)
  def _chunks(g0):
    for b in range(_NBUF):
      g = g0 + b
      nxt = g + _NBUF
      for j in range(_GPC):
        gather_copy(b, j).wait()

      @pl.when(nxt < n_chunks)
      def _prefetch_idx():
        idx_copy(nxt, b).start()

      @pl.when(g >= _NBUF)
      def _drain_out():
        out_copy(g - _NBUF, b).wait()

      _layernorm_chunk(in_vs[b], out_vs[b], g_v, b_v)
      out_copy(g, b).start()

      @pl.when(nxt < n_chunks)
      def _fire_gathers():
        idx_copy(nxt, b).wait()
        for j in range(_GPC):
          gather_copy(b, j).start()

  for b in range(_NBUF):
    out_copy(n_chunks - _NBUF + b, b).wait()


def kernel(x, table, gamma, beta):
  batch, seq = x.shape
  n = batch * seq
  d = table.shape[1]
  x2 = x.reshape(n // _GROUP, _GROUP).astype(jnp.int32)

  mesh = plsc.VectorSubcoreMesh(core_axis_name="c", subcore_axis_name="s")
  run = pl.kernel(
      _sc_body,
      out_type=jax.ShapeDtypeStruct((n, d), jnp.float32),
      mesh=mesh,
      compiler_params=pltpu.CompilerParams(
          needs_layout_passes=False, use_tc_tiling_on_sc=False),
      scratch_types=[
          pltpu.VMEM((_GPC, _GROUP), jnp.int32),
          pltpu.VMEM((_GPC, _GROUP), jnp.int32),
          pltpu.VMEM((_CHUNK, _D), jnp.float32),
          pltpu.VMEM((_CHUNK, _D), jnp.float32),
          pltpu.VMEM((_CHUNK, _D), jnp.float32),
          pltpu.VMEM((_CHUNK, _D), jnp.float32),
          pltpu.VMEM((_D,), jnp.float32),
          pltpu.VMEM((_D,), jnp.float32),
          pltpu.SemaphoreType.DMA,
          pltpu.SemaphoreType.DMA,
          pltpu.SemaphoreType.DMA,
          pltpu.SemaphoreType.DMA,
          pltpu.SemaphoreType.DMA,
          pltpu.SemaphoreType.DMA,
      ],
  )
  out = run(x2, table, gamma, beta)
  return out.reshape(batch, seq, d)
